# Initial kernel scaffold; baseline (speedup 1.0000x reference)
import jax
import jax.numpy as jnp
from jax.experimental import pallas as pl


def kernel(user_ids, history, user_table, item_table):
    raise NotImplementedError("write your pallas kernel here")


# speedup vs baseline: 1.5793x; 1.5793x over previous
import dataclasses
import functools

import jax
import jax.numpy as jnp
from jax import lax
from jax.experimental import pallas as pl
from jax.experimental.pallas import tpu as pltpu
from jax.experimental.pallas import tpu_sc as plsc

B = 16384
H = 50
HP = 64
DIM = 32
L = 16

NC = 2
NS = 16
NW = NC * NS
BPW = B // NW
CH = 32
NCHUNK = BPW // CH
IDX_PER_CHUNK = CH * H
GFULL = IDX_PER_CHUNK // 128
GREM = IDX_PER_CHUNK - GFULL * 128

_mesh = plsc.VectorSubcoreMesh(core_axis_name="c", subcore_axis_name="s")

_cp = pltpu.CompilerParams()
if "needs_layout_passes" in pltpu.CompilerParams.__dataclass_fields__:
    _cp = dataclasses.replace(_cp, needs_layout_passes=False)
if "use_tc_tiling_on_sc" in pltpu.CompilerParams.__dataclass_fields__:
    _cp = dataclasses.replace(_cp, use_tc_tiling_on_sc=False)


@functools.partial(
    pl.kernel,
    mesh=_mesh,
    compiler_params=_cp,
    out_type=jax.ShapeDtypeStruct((B, DIM), jnp.float32),
    scratch_types=[
        pltpu.VMEM((IDX_PER_CHUNK,), jnp.int32),
        pltpu.VMEM((CH * HP,), jnp.int32),
        pltpu.VMEM((CH,), jnp.int32),
        pltpu.VMEM((IDX_PER_CHUNK, DIM), jnp.float32),
        pltpu.VMEM((CH, DIM), jnp.float32),
        pltpu.VMEM((CH, DIM), jnp.float32),
        pltpu.SemaphoreType.DMA,
        pltpu.SemaphoreType.DMA,
    ],
)
def _user_repr_sc(
    hist_hbm,
    histp_hbm,
    uid_hbm,
    utab_hbm,
    itab_hbm,
    out_hbm,
    idx_v,
    idxp_v,
    uidx_v,
    rows_v,
    urows_v,
    out_v,
    gsem,
    usem,
):
    wid = lax.axis_index("s") * NC + lax.axis_index("c")
    base = wid * BPW

    @pl.loop(0, NCHUNK)
    def _chunk(c):
        rbase = base + c * CH

        pltpu.sync_copy(hist_hbm.at[pl.ds(rbase * H, IDX_PER_CHUNK)], idx_v)
        pltpu.sync_copy(histp_hbm.at[pl.ds(rbase * HP, CH * HP)], idxp_v)
        pltpu.sync_copy(uid_hbm.at[pl.ds(rbase, CH)], uidx_v)

        copies = []
        for j in range(GFULL):
            copies.append(
                pltpu.async_copy(
                    itab_hbm.at[idx_v.at[pl.ds(j * 128, 128)]],
                    rows_v.at[pl.ds(j * 128, 128)],
                    gsem,
                )
            )
        copies.append(
            pltpu.async_copy(
                itab_hbm.at[idx_v.at[pl.ds(GFULL * 128, GREM)]],
                rows_v.at[pl.ds(GFULL * 128, GREM)],
                gsem,
            )
        )
        copies.append(pltpu.async_copy(utab_hbm.at[uidx_v], urows_v, usem))
        for cp in copies:
            cp.wait()

        @pl.loop(0, CH)
        def _row(r):
            mcnt = jnp.zeros((L,), jnp.float32)
            for j in range(HP // L):
                v = idxp_v[pl.ds(r * HP + j * L, L)]
                mcnt = mcnt + jnp.where(v > 0, 1.0, 0.0).astype(jnp.float32)
            denom = jnp.broadcast_to(jnp.sum(mcnt), (L,)) + 1e-8
            recip = jnp.full((L,), 1.0, jnp.float32) / denom

            def step(h, carry):
                a0, a1 = carry
                a0 = a0 + rows_v[r * H + h, pl.ds(0, L)]
                a1 = a1 + rows_v[r * H + h, pl.ds(L, L)]
                return (a0, a1)

            zero = jnp.zeros((L,), jnp.float32)
            a0, a1 = lax.fori_loop(0, H, step, (zero, zero))

            out_v[r, pl.ds(0, L)] = urows_v[r, pl.ds(0, L)] + a0 * recip
            out_v[r, pl.ds(L, L)] = urows_v[r, pl.ds(L, L)] + a1 * recip

        pltpu.sync_copy(out_v, out_hbm.at[pl.ds(rbase, CH)])


def kernel(user_ids, history, user_table, item_table):
    user_ids = user_ids.astype(jnp.int32)
    history = history.astype(jnp.int32)
    hist_flat = history.reshape(-1)
    histp_flat = jnp.pad(history, ((0, 0), (0, HP - H))).reshape(-1)
    return _user_repr_sc(hist_flat, histp_flat, user_ids, user_table, item_table)


# speedup vs baseline: 1.8317x; 1.1599x over previous
import dataclasses
import functools

import jax
import jax.numpy as jnp
from jax import lax
from jax.experimental import pallas as pl
from jax.experimental.pallas import tpu as pltpu
from jax.experimental.pallas import tpu_sc as plsc

B = 16384
H = 50
HP = 64
DIM = 32
L = 16

NC = 2
NS = 16
NW = NC * NS
BPW = B // NW
CH = 32
NCHUNK = BPW // CH
IDX_PER_CHUNK = CH * H
GFULL = IDX_PER_CHUNK // 128
GREM = IDX_PER_CHUNK - GFULL * 128

_mesh = plsc.VectorSubcoreMesh(core_axis_name="c", subcore_axis_name="s")

_cp = pltpu.CompilerParams()
if "needs_layout_passes" in pltpu.CompilerParams.__dataclass_fields__:
    _cp = dataclasses.replace(_cp, needs_layout_passes=False)
if "use_tc_tiling_on_sc" in pltpu.CompilerParams.__dataclass_fields__:
    _cp = dataclasses.replace(_cp, use_tc_tiling_on_sc=False)


@functools.partial(
    pl.kernel,
    mesh=_mesh,
    compiler_params=_cp,
    out_type=jax.ShapeDtypeStruct((B, DIM), jnp.float32),
    scratch_types=[
        pltpu.VMEM((IDX_PER_CHUNK,), jnp.int32),
        pltpu.VMEM((CH * HP,), jnp.int32),
        pltpu.VMEM((IDX_PER_CHUNK, DIM), jnp.float32),
        pltpu.VMEM((CH, DIM), jnp.float32),
        pltpu.SemaphoreType.DMA,
    ],
)
def _item_mean_sc(hist_hbm, histp_hbm, itab_hbm, out_hbm, idx_v, idxp_v, rows_v, out_v, gsem):
    wid = lax.axis_index("s") * NC + lax.axis_index("c")
    base = wid * BPW

    @pl.loop(0, NCHUNK)
    def _chunk(c):
        rbase = base + c * CH

        pltpu.sync_copy(hist_hbm.at[pl.ds(rbase * H, IDX_PER_CHUNK)], idx_v)
        pltpu.sync_copy(histp_hbm.at[pl.ds(rbase * HP, CH * HP)], idxp_v)

        copies = []
        for j in range(GFULL):
            copies.append(
                pltpu.async_copy(
                    itab_hbm.at[idx_v.at[pl.ds(j * 128, 128)]],
                    rows_v.at[pl.ds(j * 128, 128)],
                    gsem,
                )
            )
        copies.append(
            pltpu.async_copy(
                itab_hbm.at[idx_v.at[pl.ds(GFULL * 128, GREM)]],
                rows_v.at[pl.ds(GFULL * 128, GREM)],
                gsem,
            )
        )
        for cp in copies:
            cp.wait()

        @pl.loop(0, CH)
        def _row(r):
            mcnt = jnp.zeros((L,), jnp.float32)
            for j in range(HP // L):
                v = idxp_v[pl.ds(r * HP + j * L, L)]
                mcnt = mcnt + jnp.where(v > 0, 1.0, 0.0).astype(jnp.float32)
            denom = jnp.broadcast_to(jnp.sum(mcnt), (L,)) + 1e-8
            recip = jnp.full((L,), 1.0, jnp.float32) / denom

            def step(h, carry):
                a0, a1 = carry
                a0 = a0 + rows_v[r * H + h, pl.ds(0, L)]
                a1 = a1 + rows_v[r * H + h, pl.ds(L, L)]
                return (a0, a1)

            zero = jnp.zeros((L,), jnp.float32)
            a0, a1 = lax.fori_loop(0, H, step, (zero, zero))

            out_v[r, pl.ds(0, L)] = a0 * recip
            out_v[r, pl.ds(L, L)] = a1 * recip

        pltpu.sync_copy(out_v, out_hbm.at[pl.ds(rbase, CH)])


@functools.partial(
    pl.kernel,
    mesh=_mesh,
    compiler_params=_cp,
    out_type=jax.ShapeDtypeStruct((B, DIM), jnp.float32),
    scratch_types=[
        pltpu.VMEM((BPW,), jnp.int32),
        pltpu.VMEM((BPW, DIM), jnp.float32),
        pltpu.SemaphoreType.DMA,
    ],
)
def _user_gather_sc(uid_hbm, utab_hbm, out_hbm, uidx_v, urows_v, usem):
    wid = lax.axis_index("s") * NC + lax.axis_index("c")
    base = wid * BPW
    pltpu.sync_copy(uid_hbm.at[pl.ds(base, BPW)], uidx_v)
    copies = []
    for j in range(BPW // 128):
        copies.append(
            pltpu.async_copy(
                utab_hbm.at[uidx_v.at[pl.ds(j * 128, 128)]],
                urows_v.at[pl.ds(j * 128, 128)],
                usem,
            )
        )
    for cp in copies:
        cp.wait()
    pltpu.sync_copy(urows_v, out_hbm.at[pl.ds(base, BPW)])


def _combine_body(a_ref, b_ref, o_ref):
    o_ref[...] = a_ref[...] + b_ref[...]


_combine_tc = pl.pallas_call(
    _combine_body,
    out_shape=jax.ShapeDtypeStruct((B, DIM), jnp.float32),
    grid=(8,),
    in_specs=[
        pl.BlockSpec((B // 8, DIM), lambda i: (i, 0)),
        pl.BlockSpec((B // 8, DIM), lambda i: (i, 0)),
    ],
    out_specs=pl.BlockSpec((B // 8, DIM), lambda i: (i, 0)),
)


def kernel(user_ids, history, user_table, item_table):
    user_ids = user_ids.astype(jnp.int32)
    history = history.astype(jnp.int32)
    hist_flat = history.reshape(-1)
    histp_flat = jnp.pad(history, ((0, 0), (0, HP - H))).reshape(-1)
    hist_mean = _item_mean_sc(hist_flat, histp_flat, item_table)
    user_rows = _user_gather_sc(user_ids, user_table)
    return _combine_tc(user_rows, hist_mean)
